# baseline (speedup 1.0000x reference)
import jax
import jax.numpy as jnp
from jax import lax
from jax.experimental import pallas as pl
from jax.experimental.pallas import tpu as pltpu
from jax.experimental.pallas import tpu_sc as plsc

NUM_CORES = 2
NUM_SUBCORES = 16
NUM_WORKERS = NUM_CORES * NUM_SUBCORES
LANES = 16
BATCH = 16384
RANK = 32
BPW = BATCH // NUM_WORKERS
IDX_CHUNK = 128
NCHUNK = BPW // IDX_CHUNK


def _gmf_body(user_h, item_h, eu_h, ei_h, wb_h, out_h,
              idxu_v, idxi_v, ru_v, ri_v, pt_v, out_v, wb_v, sem):
    wid = lax.axis_index("s") * NUM_CORES + lax.axis_index("c")
    base = wid * BPW

    pltpu.sync_copy(wb_h, wb_v)
    for c in range(NCHUNK):
        pltpu.sync_copy(user_h.at[pl.ds(base + c * IDX_CHUNK, IDX_CHUNK)],
                        idxu_v.at[c])
        pltpu.sync_copy(item_h.at[pl.ds(base + c * IDX_CHUNK, IDX_CHUNK)],
                        idxi_v.at[c])

    copies = []
    for c in range(NCHUNK):
        copies.append(pltpu.async_copy(
            eu_h.at[idxu_v.at[c]], ru_v.at[pl.ds(c * IDX_CHUNK, IDX_CHUNK)], sem))
        copies.append(pltpu.async_copy(
            ei_h.at[idxi_v.at[c]], ri_v.at[pl.ds(c * IDX_CHUNK, IDX_CHUNK)], sem))
    for cp in copies:
        cp.wait()

    w0 = wb_v[pl.ds(0, LANES)]
    w1 = wb_v[pl.ds(LANES, LANES)]
    bv = wb_v[pl.ds(2 * LANES, LANES)]
    lane_iota = lax.broadcasted_iota(jnp.int32, (LANES,), 0)

    UNROLL = 8

    def stage_a(ib, carry):
        for k in range(UNROLL):
            i = ib * UNROLL + k
            p = (ru_v[i, pl.ds(0, LANES)] * ri_v[i, pl.ds(0, LANES)] * w0
                 + ru_v[i, pl.ds(LANES, LANES)] * ri_v[i, pl.ds(LANES, LANES)] * w1)
            plsc.store_scatter(pt_v, [lane_iota * BPW + i], p)
        return carry

    lax.fori_loop(0, BPW // UNROLL, stage_a, 0)

    def stage_b(bi, carry):
        acc = bv
        for l in range(LANES):
            acc = acc + pt_v[pl.ds(l * BPW + bi * LANES, LANES)]
        out_v[pl.ds(bi * LANES, LANES)] = acc
        return carry

    lax.fori_loop(0, BPW // LANES, stage_b, 0)

    pltpu.sync_copy(out_v, out_h.at[pl.ds(base, BPW)])


def kernel(user, item, embed_user, embed_item, W, b):
    wb = jnp.concatenate(
        [W.reshape(RANK), jnp.full((LANES,), b.reshape(-1)[0], jnp.float32)]
    ).astype(jnp.float32)

    run = pl.kernel(
        _gmf_body,
        out_type=jax.ShapeDtypeStruct((BATCH,), jnp.float32),
        mesh=plsc.VectorSubcoreMesh(core_axis_name="c", subcore_axis_name="s"),
        compiler_params=pltpu.CompilerParams(
            needs_layout_passes=False, use_tc_tiling_on_sc=False),
        scratch_types=[
            pltpu.VMEM((NCHUNK, IDX_CHUNK), jnp.int32),
            pltpu.VMEM((NCHUNK, IDX_CHUNK), jnp.int32),
            pltpu.VMEM((BPW, RANK), jnp.float32),
            pltpu.VMEM((BPW, RANK), jnp.float32),
            pltpu.VMEM((LANES * BPW,), jnp.float32),
            pltpu.VMEM((BPW,), jnp.float32),
            pltpu.VMEM((RANK + LANES,), jnp.float32),
            pltpu.SemaphoreType.DMA,
        ],
    )
    return run(user.astype(jnp.int32), item.astype(jnp.int32),
               embed_user, embed_item, wb)
